# Initial kernel scaffold; baseline (speedup 1.0000x reference)
import jax
import jax.numpy as jnp
from jax.experimental import pallas as pl


def kernel(word, tag, pos1, pos2, word_table, tag_table, pos1_table, pos2_table):
    raise NotImplementedError("write your pallas kernel here")


# speedup vs baseline: 3.7407x; 3.7407x over previous
import jax
import jax.numpy as jnp
from jax import lax
from jax.experimental import pallas as pl
from jax.experimental.pallas import tpu as pltpu
from jax.experimental.pallas import tpu_sc as plsc

B = 1024
L = 200
N = B * L
WORD_DIM = 128
SMALL_DIM = 16
OUT_DIM = WORD_DIM + 3 * SMALL_DIM

NC = 2
NS = 16
NW = NC * NS
N_PER_W = N // NW
CHUNK = 128
N_CHUNKS = N_PER_W // CHUNK


def _sc_body(word_hbm, tag_hbm, pos1_hbm, pos2_hbm,
             word_tbl, tag_tbl, pos1_tbl, pos2_tbl,
             out_hbm,
             widx, tidx, p1idx, p2idx,
             wbuf, tbuf, p1buf, p2buf, gsem, osem):
  wid = lax.axis_index("s") * NC + lax.axis_index("c")
  wbase = wid * N_PER_W

  def chunk_body(i, _):
    base = wbase + i * CHUNK
    pltpu.sync_copy(word_hbm.at[pl.ds(base, CHUNK)], widx)
    pltpu.sync_copy(tag_hbm.at[pl.ds(base, CHUNK)], tidx)
    pltpu.sync_copy(pos1_hbm.at[pl.ds(base, CHUNK)], p1idx)
    pltpu.sync_copy(pos2_hbm.at[pl.ds(base, CHUNK)], p2idx)
    cw = pltpu.async_copy(word_tbl.at[widx], wbuf, gsem)
    ct = pltpu.async_copy(tag_tbl.at[tidx], tbuf, gsem)
    c1 = pltpu.async_copy(pos1_tbl.at[p1idx], p1buf, gsem)
    c2 = pltpu.async_copy(pos2_tbl.at[p2idx], p2buf, gsem)
    cw.wait(); ct.wait(); c1.wait(); c2.wait()
    ow = pltpu.async_copy(wbuf, out_hbm.at[pl.ds(base, CHUNK), pl.ds(0, WORD_DIM)], osem)
    ot = pltpu.async_copy(tbuf, out_hbm.at[pl.ds(base, CHUNK), pl.ds(128, SMALL_DIM)], osem)
    o1 = pltpu.async_copy(p1buf, out_hbm.at[pl.ds(base, CHUNK), pl.ds(144, SMALL_DIM)], osem)
    o2 = pltpu.async_copy(p2buf, out_hbm.at[pl.ds(base, CHUNK), pl.ds(160, SMALL_DIM)], osem)
    ow.wait(); ot.wait(); o1.wait(); o2.wait()
    return ()

  lax.fori_loop(0, N_CHUNKS, chunk_body, ())


@jax.jit
def _embed(word, tag, pos1, pos2, word_tbl, tag_tbl, pos1_tbl, pos2_tbl):
  mesh = plsc.VectorSubcoreMesh(core_axis_name="c", subcore_axis_name="s")
  f = pl.kernel(
      _sc_body,
      out_type=jax.ShapeDtypeStruct((N, OUT_DIM), jnp.float32),
      mesh=mesh,
      scratch_types=[
          pltpu.VMEM((CHUNK,), jnp.int32),
          pltpu.VMEM((CHUNK,), jnp.int32),
          pltpu.VMEM((CHUNK,), jnp.int32),
          pltpu.VMEM((CHUNK,), jnp.int32),
          pltpu.VMEM((CHUNK, WORD_DIM), jnp.float32),
          pltpu.VMEM((CHUNK, SMALL_DIM), jnp.float32),
          pltpu.VMEM((CHUNK, SMALL_DIM), jnp.float32),
          pltpu.VMEM((CHUNK, SMALL_DIM), jnp.float32),
          pltpu.SemaphoreType.DMA,
          pltpu.SemaphoreType.DMA,
      ],
      compiler_params=pltpu.CompilerParams(use_tc_tiling_on_sc=False),
  )
  return f(word, tag, pos1, pos2, word_tbl, tag_tbl, pos1_tbl, pos2_tbl)


def kernel(word, tag, pos1, pos2, word_table, tag_table, pos1_table, pos2_table):
  word = word.reshape(N).astype(jnp.int32)
  tag = tag.reshape(N).astype(jnp.int32)
  pos1 = pos1.reshape(N).astype(jnp.int32)
  pos2 = pos2.reshape(N).astype(jnp.int32)
  out = _embed(word, tag, pos1, pos2,
               word_table, tag_table, pos1_table, pos2_table)
  return out.reshape(B, L, OUT_DIM)
